# baseline (device time: 81151 ns/iter reference)
import numpy as np

import jax
import jax.numpy as jnp
from jax import lax
from jax.experimental import pallas as pl
from jax.experimental.pallas import tpu as pltpu

N_DEV = 8
B, SQ, D = 2, 512, 1024
T = B * SQ
HL, DH = 8, 128
SCALE = 0.08838834764831843
WIRE = jnp.bfloat16

COLS = [(0, 384), (384, 768), (768, 1024)]
SBITS = [[2, 1, 0], [1, 0, 2], [0, 2, 1]]
RS_ROWS = [512, 256, 128]
AG_ROWS = [128, 256, 512]


def _rope_tables():
    inv = 1.0 / (10000.0 ** (np.arange(0, DH, 2) / DH))
    pos = np.arange(SQ)[:, None] * inv[None, :]
    ck, sk = np.cos(pos), np.sin(pos)
    cos2 = np.concatenate([ck, ck], axis=1)
    sin2 = np.concatenate([-sk, sk], axis=1)
    cos_t = np.concatenate([cos2, cos2], axis=0)
    sin_t = np.concatenate([sin2, sin2], axis=0)
    return cos_t.astype(np.float32), sin_t.astype(np.float32)


_COS, _SIN = _rope_tables()


def _deinterleave_cols(w):
    return w.reshape(D, HL, DH // 2, 2).transpose(0, 1, 3, 2).reshape(D, HL * DH)


def _vid(d):
    return d ^ ((d >> 1) & 1)


def _body(*refs):
    (x_ref, wq_ref, wk_ref, wv_ref, wo_ref,
     cosq_ref, sinq_ref, cos_ref, sin_ref, out_ref) = refs[:10]
    qb, kb, vb = refs[10:13]
    sbufs = refs[13:31]
    rbufs = refs[31:49]
    ssem, rsem = refs[49], refs[50]

    me = lax.axis_index("i")
    vm = _vid(me)
    bits = [vm & 1, (vm >> 1) & 1, (vm >> 2) & 1]

    barrier_sem = pltpu.get_barrier_semaphore()
    for b in range(3):
        pl.semaphore_signal(barrier_sem, inc=1, device_id=(_vid(vm ^ (1 << b)),),
                            device_id_type=pl.DeviceIdType.MESH)
    pl.semaphore_wait(barrier_sem, 3)

    def exchange(slot, p, bitpos):
        return pltpu.make_async_remote_copy(
            src_ref=sbufs[slot * 3 + p], dst_ref=rbufs[slot * 3 + p],
            send_sem=ssem.at[slot * 3 + p], recv_sem=rsem.at[slot * 3 + p],
            device_id=(_vid(vm ^ (1 << bitpos)),),
            device_id_type=pl.DeviceIdType.MESH,
        )

    qb[:, :] = jnp.dot(x_ref[:, :], wq_ref[:, :], preferred_element_type=jnp.float32)
    kb[:, :] = jnp.dot(x_ref[:, :], wk_ref[:, :], preferred_element_type=jnp.float32)
    vb[:, :] = jnp.dot(x_ref[:, :], wv_ref[:, :], preferred_element_type=jnp.float32)
    for h in range(HL):
        cs = slice(h * DH, (h + 1) * DH)
        q = qb[:, cs]
        qb[:, cs] = q * cosq_ref[:, :] + pltpu.roll(q, 64, 1) * sinq_ref[:, :]
        k = kb[:, cs]
        kb[:, cs] = k * cos_ref[:, :] + pltpu.roll(k, 64, 1) * sin_ref[:, :]

    def compute_batch(b):
        rows = slice(b * SQ, (b + 1) * SQ)
        for h in range(HL):
            cs = slice(h * DH, (h + 1) * DH)
            s = lax.dot_general(qb[rows, cs], kb[rows, cs],
                                (((1,), (1,)), ((), ())),
                                preferred_element_type=jnp.float32)
            e = jnp.exp(s)
            den = jnp.sum(e, axis=1, keepdims=True)
            vb[rows, cs] = jnp.dot(e, vb[rows, cs],
                                   preferred_element_type=jnp.float32) / den
        out_ref[rows, :] = jnp.dot(vb[rows, :], wo_ref[:, :],
                                   preferred_element_type=jnp.float32)

    ex0, lo_sends, keeps0 = [], [], []
    for p in range(3):
        bp = bits[SBITS[p][0]]
        lo_sends.append((1 - bp) * 512)
        keeps0.append(bp * 512)
        ex0.append(exchange(0, p, SBITS[p][0]))

    for b in range(B):
        compute_batch(b)
        for p in range(3):
            c0, c1 = COLS[p]

            @pl.when(lo_sends[p] == b * 512)
            def _(p=p, c0=c0, c1=c1, b=b):
                sbufs[p][:, :] = out_ref[b * 512:(b + 1) * 512, c0:c1].astype(WIRE)
                ex0[p].start()

    for p in range(3):
        c0, c1 = COLS[p]
        ex0[p].wait()
        out_ref[pl.ds(keeps0[p], 512), c0:c1] = (
            out_ref[pl.ds(keeps0[p], 512), c0:c1]
            + rbufs[p][:, :].astype(jnp.float32)
        )

    los = keeps0
    for k in range(1, 3):
        half = RS_ROWS[k]
        exs, keeps = [], []
        for p in range(3):
            c0, c1 = COLS[p]
            bp = bits[SBITS[p][k]]
            lo_send = los[p] + (1 - bp) * half
            lo_keep = los[p] + bp * half
            sbufs[k * 3 + p][:, :] = out_ref[pl.ds(lo_send, half), c0:c1].astype(WIRE)
            ex = exchange(k, p, SBITS[p][k])
            ex.start()
            exs.append(ex)
            keeps.append(lo_keep)
        for p in range(3):
            c0, c1 = COLS[p]
            exs[p].wait()
            out_ref[pl.ds(keeps[p], half), c0:c1] = (
                out_ref[pl.ds(keeps[p], half), c0:c1]
                + rbufs[k * 3 + p][:, :].astype(jnp.float32)
            )
        los = keeps

    for j in range(3):
        ln = AG_ROWS[j]
        slot = 3 + j
        exs, plos, nlos = [], [], []
        for p in range(3):
            c0, c1 = COLS[p]
            bitpos = SBITS[p][2 - j]
            bp = bits[bitpos]
            sbufs[slot * 3 + p][:, :] = out_ref[pl.ds(los[p], ln), c0:c1].astype(WIRE)
            ex = exchange(slot, p, bitpos)
            ex.start()
            exs.append(ex)
            plos.append(los[p] + (1 - 2 * bp) * ln)
            nlos.append(los[p] - bp * ln)
        for p in range(3):
            c0, c1 = COLS[p]
            exs[p].wait()
            out_ref[pl.ds(plos[p], ln), c0:c1] = rbufs[slot * 3 + p][:, :].astype(
                jnp.float32)
        los = nlos


def kernel(x, Wq, Wk, Wv, Wo):
    x2 = x.reshape(T, D)
    wq = _deinterleave_cols(Wq)
    wk = _deinterleave_cols(Wk)
    cos_t = jnp.asarray(_COS)
    sin_t = jnp.asarray(_SIN)
    cosq_t = jnp.asarray(_COS * np.float32(SCALE))
    sinq_t = jnp.asarray(_SIN * np.float32(SCALE))

    comm_shapes = []
    for rows in RS_ROWS + AG_ROWS:
        for (c0, c1) in COLS:
            comm_shapes.append(pltpu.VMEM((rows, c1 - c0), WIRE))

    out = pl.pallas_call(
        _body,
        out_shape=jax.ShapeDtypeStruct((T, D), jnp.float32),
        in_specs=[pl.BlockSpec(memory_space=pltpu.VMEM)] * 9,
        out_specs=pl.BlockSpec(memory_space=pltpu.VMEM),
        scratch_shapes=(
            [pltpu.VMEM((T, HL * DH), jnp.float32)] * 3
            + comm_shapes
            + comm_shapes
            + [pltpu.SemaphoreType.DMA((18,)),
               pltpu.SemaphoreType.DMA((18,))]
        ),
        compiler_params=pltpu.CompilerParams(
            collective_id=0, vmem_limit_bytes=100 * 1024 * 1024
        ),
    )(x2, wq, wk, Wv, Wo, cosq_t, sinq_t, cos_t, sin_t)
    return out.reshape(B, SQ, D)


# device time: 80956 ns/iter; 1.0024x vs baseline; 1.0024x over previous
import numpy as np

import jax
import jax.numpy as jnp
from jax import lax
from jax.experimental import pallas as pl
from jax.experimental.pallas import tpu as pltpu

N_DEV = 8
B, SQ, D = 2, 512, 1024
T = B * SQ
HL, DH = 8, 128
SCALE = 0.08838834764831843
WIRE = jnp.bfloat16

COLS = [(0, 384), (384, 768), (768, 1024)]
SBITS = [[2, 1, 0], [1, 0, 2], [0, 2, 1]]
RS_ROWS = [512, 256, 128]
AG_ROWS = [128, 256, 512]


def _rope_tables():
    inv = 1.0 / (10000.0 ** (np.arange(0, DH, 2) / DH))
    pos = np.arange(SQ)[:, None] * inv[None, :]
    ck, sk = np.cos(pos), np.sin(pos)
    cos2 = np.concatenate([ck, ck], axis=1)
    sin2 = np.concatenate([-sk, sk], axis=1)
    cos_t = np.concatenate([cos2, cos2], axis=0)
    sin_t = np.concatenate([sin2, sin2], axis=0)
    return cos_t.astype(np.float32), sin_t.astype(np.float32)


_COS, _SIN = _rope_tables()


def _deinterleave_cols(w):
    return w.reshape(D, HL, DH // 2, 2).transpose(0, 1, 3, 2).reshape(D, HL * DH)


def _vid(d):
    return d ^ ((d >> 1) & 1)


def _body(*refs):
    (x_ref, wq_ref, wk_ref, wv_ref, wo_ref,
     cosq_ref, sinq_ref, cos_ref, sin_ref, out_ref) = refs[:10]
    qb, kb, vb = refs[10:13]
    pbs = refs[13:16]
    rbufs = refs[16:25]
    ssem, rsem = refs[25], refs[26]

    me = lax.axis_index("i")
    vm = _vid(me)
    bits = [vm & 1, (vm >> 1) & 1, (vm >> 2) & 1]

    barrier_sem = pltpu.get_barrier_semaphore()
    for b in range(3):
        pl.semaphore_signal(barrier_sem, inc=1, device_id=(_vid(vm ^ (1 << b)),),
                            device_id_type=pl.DeviceIdType.MESH)
    pl.semaphore_wait(barrier_sem, 3)

    def rdma(idx, src, dst, bitpos):
        return pltpu.make_async_remote_copy(
            src_ref=src, dst_ref=dst,
            send_sem=ssem.at[idx], recv_sem=rsem.at[idx],
            device_id=(_vid(vm ^ (1 << bitpos)),),
            device_id_type=pl.DeviceIdType.MESH,
        )

    qb[:, :] = jnp.dot(x_ref[:, :], wq_ref[:, :], preferred_element_type=jnp.float32)
    kb[:, :] = jnp.dot(x_ref[:, :], wk_ref[:, :], preferred_element_type=jnp.float32)
    vb[:, :] = jnp.dot(x_ref[:, :], wv_ref[:, :], preferred_element_type=jnp.float32)
    for h in range(HL):
        cs = slice(h * DH, (h + 1) * DH)
        q = qb[:, cs]
        qb[:, cs] = q * cosq_ref[:, :] + pltpu.roll(q, 64, 1) * sinq_ref[:, :]
        k = kb[:, cs]
        kb[:, cs] = k * cos_ref[:, :] + pltpu.roll(k, 64, 1) * sin_ref[:, :]

    for b in range(B):
        rows = slice(b * SQ, (b + 1) * SQ)
        for h in range(HL):
            cs = slice(h * DH, (h + 1) * DH)
            s = lax.dot_general(qb[rows, cs], kb[rows, cs],
                                (((1,), (1,)), ((), ())),
                                preferred_element_type=jnp.float32)
            e = jnp.exp(s)
            den = jnp.sum(e, axis=1, keepdims=True)
            vb[rows, cs] = jnp.dot(e, vb[rows, cs],
                                   preferred_element_type=jnp.float32) / den
        for p in range(3):
            c0, c1 = COLS[p]
            pbs[p][rows, :] = jnp.dot(vb[rows, :], wo_ref[:, c0:c1],
                                      preferred_element_type=jnp.float32
                                      ).astype(WIRE)

    los = [0, 0, 0]
    for k in range(3):
        half = RS_ROWS[k]
        exs, keeps = [], []
        for p in range(3):
            bp = bits[SBITS[p][k]]
            lo_send = los[p] + (1 - bp) * half
            lo_keep = los[p] + bp * half
            ex = rdma(k * 3 + p, pbs[p].at[pl.ds(lo_send, half)],
                      rbufs[k * 3 + p], SBITS[p][k])
            ex.start()
            exs.append(ex)
            keeps.append(lo_keep)
        for p in range(3):
            exs[p].wait()
            pbs[p][pl.ds(keeps[p], half), :] = (
                pbs[p][pl.ds(keeps[p], half), :] + rbufs[k * 3 + p][:, :]
            )
        los = keeps

    for j in range(3):
        ln = AG_ROWS[j]
        exs, nlos = [], []
        for p in range(3):
            bitpos = SBITS[p][2 - j]
            bp = bits[bitpos]
            ex = rdma(9 + j * 3 + p, pbs[p].at[pl.ds(los[p], ln)],
                      pbs[p].at[pl.ds(los[p], ln)], bitpos)
            ex.start()
            exs.append(ex)
            nlos.append(los[p] - bp * ln)
        for p in range(3):
            exs[p].wait()
        los = nlos

    for p in range(3):
        c0, c1 = COLS[p]
        out_ref[:, c0:c1] = pbs[p][:, :].astype(jnp.float32)


def kernel(x, Wq, Wk, Wv, Wo):
    x2 = x.reshape(T, D)
    wq = _deinterleave_cols(Wq)
    wk = _deinterleave_cols(Wk)
    cos_t = jnp.asarray(_COS)
    sin_t = jnp.asarray(_SIN)
    cosq_t = jnp.asarray(_COS * np.float32(SCALE))
    sinq_t = jnp.asarray(_SIN * np.float32(SCALE))

    comm_shapes = [pltpu.VMEM((T, c1 - c0), WIRE) for (c0, c1) in COLS]
    for rows in RS_ROWS:
        for (c0, c1) in COLS:
            comm_shapes.append(pltpu.VMEM((rows, c1 - c0), WIRE))

    out = pl.pallas_call(
        _body,
        out_shape=jax.ShapeDtypeStruct((T, D), jnp.float32),
        in_specs=[pl.BlockSpec(memory_space=pltpu.VMEM)] * 9,
        out_specs=pl.BlockSpec(memory_space=pltpu.VMEM),
        scratch_shapes=(
            [pltpu.VMEM((T, HL * DH), jnp.float32)] * 3
            + comm_shapes
            + [pltpu.SemaphoreType.DMA((18,)),
               pltpu.SemaphoreType.DMA((18,))]
        ),
        compiler_params=pltpu.CompilerParams(
            collective_id=0, vmem_limit_bytes=100 * 1024 * 1024
        ),
    )(x2, wq, wk, Wv, Wo, cosq_t, sinq_t, cos_t, sin_t)
    return out.reshape(B, SQ, D)


# device time: 80342 ns/iter; 1.0101x vs baseline; 1.0076x over previous
import numpy as np

import jax
import jax.numpy as jnp
from jax import lax
from jax.experimental import pallas as pl
from jax.experimental.pallas import tpu as pltpu

N_DEV = 8
B, SQ, D = 2, 512, 1024
T = B * SQ
HL, DH = 8, 128
SCALE = 0.08838834764831843
WIRE = jnp.bfloat16

COLS = [(0, 384), (384, 768), (768, 1024)]
SBITS = [[2, 1, 0], [1, 0, 2], [0, 2, 1]]
RS_ROWS = [512, 256, 128]
AG_ROWS = [128, 256, 512]


def _rope_tables():
    inv = 1.0 / (10000.0 ** (np.arange(0, DH, 2) / DH))
    pos = np.arange(SQ)[:, None] * inv[None, :]
    ck, sk = np.cos(pos), np.sin(pos)
    cos2 = np.concatenate([ck, ck], axis=1)
    sin2 = np.concatenate([-sk, sk], axis=1)
    cos_t = np.concatenate([cos2, cos2], axis=0)
    sin_t = np.concatenate([sin2, sin2], axis=0)
    return cos_t.astype(np.float32), sin_t.astype(np.float32)


_COS, _SIN = _rope_tables()


def _deinterleave_cols(w):
    return w.reshape(D, HL, DH // 2, 2).transpose(0, 1, 3, 2).reshape(D, HL * DH)


def _vid(d):
    return d ^ ((d >> 1) & 1)


def _body(*refs):
    (x_ref, wq_ref, wk_ref, wv_ref, wo_ref,
     cosq_ref, sinq_ref, cos_ref, sin_ref, out_ref) = refs[:10]
    qb, kb, vb = refs[10:13]
    sbufs = refs[13:31]
    rbufs = refs[31:49]
    ssem, rsem = refs[49], refs[50]

    me = lax.axis_index("i")
    vm = _vid(me)
    bits = [vm & 1, (vm >> 1) & 1, (vm >> 2) & 1]

    barrier_sem = pltpu.get_barrier_semaphore()
    for b in range(3):
        pl.semaphore_signal(barrier_sem, inc=1, device_id=(_vid(vm ^ (1 << b)),),
                            device_id_type=pl.DeviceIdType.MESH)
    pl.semaphore_wait(barrier_sem, 3)

    def exchange(slot, p, bitpos):
        return pltpu.make_async_remote_copy(
            src_ref=sbufs[slot * 3 + p], dst_ref=rbufs[slot * 3 + p],
            send_sem=ssem.at[slot * 3 + p], recv_sem=rsem.at[slot * 3 + p],
            device_id=(_vid(vm ^ (1 << bitpos)),),
            device_id_type=pl.DeviceIdType.MESH,
        )

    qb[:, :] = jnp.dot(x_ref[:, :], wq_ref[:, :], preferred_element_type=jnp.float32)
    kb[:, :] = jnp.dot(x_ref[:, :], wk_ref[:, :], preferred_element_type=jnp.float32)
    vb[:, :] = jnp.dot(x_ref[:, :], wv_ref[:, :], preferred_element_type=jnp.float32)
    for h in range(HL):
        cs = slice(h * DH, (h + 1) * DH)
        q = qb[:, cs]
        qb[:, cs] = q * cosq_ref[:, :] + pltpu.roll(q, 64, 1) * sinq_ref[:, :]
        k = kb[:, cs]
        kb[:, cs] = k * cos_ref[:, :] + pltpu.roll(k, 64, 1) * sin_ref[:, :]

    for b in range(B):
        rows = slice(b * SQ, (b + 1) * SQ)
        for h in range(HL):
            cs = slice(h * DH, (h + 1) * DH)
            s = lax.dot_general(qb[rows, cs], kb[rows, cs],
                                (((1,), (1,)), ((), ())),
                                preferred_element_type=jnp.float32)
            e = jnp.exp(s)
            den = jnp.sum(e, axis=1, keepdims=True)
            vb[rows, cs] = jnp.dot(e, vb[rows, cs],
                                   preferred_element_type=jnp.float32) / den
        out_ref[rows, :] = jnp.dot(vb[rows, :], wo_ref[:, :],
                                   preferred_element_type=jnp.float32)

    los = [0, 0, 0]
    for k in range(3):
        half = RS_ROWS[k]
        exs, keeps = [], []
        for p in range(3):
            c0, c1 = COLS[p]
            bp = bits[SBITS[p][k]]
            lo_send = los[p] + (1 - bp) * half
            lo_keep = los[p] + bp * half
            sbufs[k * 3 + p][:, :] = out_ref[pl.ds(lo_send, half), c0:c1].astype(WIRE)
            ex = exchange(k, p, SBITS[p][k])
            ex.start()
            exs.append(ex)
            keeps.append(lo_keep)
        for p in range(3):
            c0, c1 = COLS[p]
            exs[p].wait()
            out_ref[pl.ds(keeps[p], half), c0:c1] = (
                out_ref[pl.ds(keeps[p], half), c0:c1]
                + rbufs[k * 3 + p][:, :].astype(jnp.float32)
            )
        los = keeps

    for j in range(3):
        ln = AG_ROWS[j]
        slot = 3 + j
        exs, plos, nlos = [], [], []
        for p in range(3):
            c0, c1 = COLS[p]
            bitpos = SBITS[p][2 - j]
            bp = bits[bitpos]
            sbufs[slot * 3 + p][:, :] = out_ref[pl.ds(los[p], ln), c0:c1].astype(WIRE)
            ex = exchange(slot, p, bitpos)
            ex.start()
            exs.append(ex)
            plos.append(los[p] + (1 - 2 * bp) * ln)
            nlos.append(los[p] - bp * ln)
        for p in range(3):
            c0, c1 = COLS[p]
            exs[p].wait()
            out_ref[pl.ds(plos[p], ln), c0:c1] = rbufs[slot * 3 + p][:, :].astype(
                jnp.float32)
        los = nlos


def kernel(x, Wq, Wk, Wv, Wo):
    x2 = x.reshape(T, D)
    wq = _deinterleave_cols(Wq)
    wk = _deinterleave_cols(Wk)
    cos_t = jnp.asarray(_COS)
    sin_t = jnp.asarray(_SIN)
    cosq_t = jnp.asarray(_COS * np.float32(SCALE))
    sinq_t = jnp.asarray(_SIN * np.float32(SCALE))

    comm_shapes = []
    for rows in RS_ROWS + AG_ROWS:
        for (c0, c1) in COLS:
            comm_shapes.append(pltpu.VMEM((rows, c1 - c0), WIRE))

    out = pl.pallas_call(
        _body,
        out_shape=jax.ShapeDtypeStruct((T, D), jnp.float32),
        in_specs=[pl.BlockSpec(memory_space=pltpu.VMEM)] * 9,
        out_specs=pl.BlockSpec(memory_space=pltpu.VMEM),
        scratch_shapes=(
            [pltpu.VMEM((T, HL * DH), jnp.float32)] * 3
            + comm_shapes
            + comm_shapes
            + [pltpu.SemaphoreType.DMA((18,)),
               pltpu.SemaphoreType.DMA((18,))]
        ),
        compiler_params=pltpu.CompilerParams(
            collective_id=0, vmem_limit_bytes=100 * 1024 * 1024
        ),
    )(x2, wq, wk, Wv, Wo, cosq_t, sinq_t, cos_t, sin_t)
    return out.reshape(B, SQ, D)
